# Initial kernel scaffold; baseline (speedup 1.0000x reference)
import jax
import jax.numpy as jnp
from jax.experimental import pallas as pl


def kernel(input_ids, word_embeddings):
    raise NotImplementedError("write your pallas kernel here")


# speedup vs baseline: 1.5362x; 1.5362x over previous
import functools

import jax
import jax.numpy as jnp
from jax import lax
from jax.experimental import pallas as pl
from jax.experimental.pallas import tpu as pltpu
from jax.experimental.pallas import tpu_sc as plsc

HIDDEN = 1024
BATCH = 8192

_info = plsc.get_sparse_core_info()
NC, NS = _info.num_cores, _info.num_subcores
NW = NC * NS
B_PER_W = BATCH // NW
CHUNK = 32
NCHUNK = B_PER_W // CHUNK

_mesh = plsc.VectorSubcoreMesh(core_axis_name="c", subcore_axis_name="s")


@functools.partial(
    pl.kernel,
    mesh=_mesh,
    out_type=jax.ShapeDtypeStruct((BATCH, HIDDEN), jnp.float32),
    scratch_types=[
        pltpu.VMEM((NCHUNK, CHUNK), jnp.int32),
        pltpu.VMEM((CHUNK, HIDDEN), jnp.float32),
        pltpu.VMEM((CHUNK, HIDDEN), jnp.float32),
        pltpu.SemaphoreType.DMA,
        pltpu.SemaphoreType.DMA,
        pltpu.SemaphoreType.DMA,
        pltpu.SemaphoreType.DMA,
    ],
)
def _sc_gather(idx_hbm, table_hbm, out_hbm, idx_v, buf0, buf1,
               gsem0, gsem1, osem0, osem1):
    wid = lax.axis_index("s") * NC + lax.axis_index("c")
    base = wid * B_PER_W
    bufs = (buf0, buf1)
    gsems = (gsem0, gsem1)
    osems = (osem0, osem1)

    pltpu.sync_copy(idx_hbm.at[wid], idx_v)

    def start_gather(g):
        b = g % 2
        return pltpu.async_copy(table_hbm.at[idx_v.at[g]], bufs[b], gsems[b])

    gat = [None, None]
    outs = [None, None]
    gat[0] = start_gather(0)
    for g in range(NCHUNK):
        b = g % 2
        nb = (g + 1) % 2
        if g + 1 < NCHUNK:
            if outs[nb] is not None:
                outs[nb].wait()
            gat[nb] = start_gather(g + 1)
        gat[b].wait()
        outs[b] = pltpu.async_copy(
            bufs[b], out_hbm.at[pl.ds(base + g * CHUNK, CHUNK)], osems[b])
    outs[(NCHUNK - 2) % 2].wait()
    outs[(NCHUNK - 1) % 2].wait()


def kernel(input_ids, word_embeddings):
    ids = input_ids.reshape(NW, NCHUNK, CHUNK).astype(jnp.int32)
    out = _sc_gather(ids, word_embeddings)
    return out.reshape(input_ids.shape + (word_embeddings.shape[-1],))


# speedup vs baseline: 1.5656x; 1.0191x over previous
import functools

import jax
import jax.numpy as jnp
from jax import lax
from jax.experimental import pallas as pl
from jax.experimental.pallas import tpu as pltpu
from jax.experimental.pallas import tpu_sc as plsc

HIDDEN = 1024
BATCH = 8192

_info = plsc.get_sparse_core_info()
NC, NS = _info.num_cores, _info.num_subcores
NW = NC * NS
B_PER_W = BATCH // NW
CHUNK = 32
NCHUNK = B_PER_W // CHUNK
NBUF = 3

_mesh = plsc.VectorSubcoreMesh(core_axis_name="c", subcore_axis_name="s")


@functools.partial(
    pl.kernel,
    mesh=_mesh,
    out_type=jax.ShapeDtypeStruct((BATCH, HIDDEN), jnp.float32),
    scratch_types=(
        [pltpu.VMEM((NCHUNK, CHUNK), jnp.int32)]
        + [pltpu.VMEM((CHUNK, HIDDEN), jnp.float32)] * NBUF
        + [pltpu.SemaphoreType.DMA] * (2 * NBUF)
    ),
)
def _sc_gather(idx_hbm, table_hbm, out_hbm, idx_v, *rest):
    bufs = rest[:NBUF]
    gsems = rest[NBUF:2 * NBUF]
    osems = rest[2 * NBUF:]
    wid = lax.axis_index("s") * NC + lax.axis_index("c")
    base = wid * B_PER_W

    pltpu.sync_copy(idx_hbm.at[wid], idx_v)

    def start_gather(g):
        b = g % NBUF
        return pltpu.async_copy(table_hbm.at[idx_v.at[g]], bufs[b], gsems[b])

    gat = [None] * NBUF
    outs = [None] * NBUF
    for j in range(min(NBUF, NCHUNK)):
        gat[j] = start_gather(j)
    sched = None
    for g in range(NCHUNK):
        b = g % NBUF
        if sched is not None:
            sb, sg = sched
            outs[sb].wait()
            gat[sb] = start_gather(sg)
            outs[sb] = None
            sched = None
        gat[b].wait()
        outs[b] = pltpu.async_copy(
            bufs[b], out_hbm.at[pl.ds(base + g * CHUNK, CHUNK)], osems[b])
        if g + NBUF < NCHUNK:
            sched = (b, g + NBUF)
    for o in outs:
        if o is not None:
            o.wait()


def kernel(input_ids, word_embeddings):
    ids = input_ids.reshape(NW, NCHUNK, CHUNK).astype(jnp.int32)
    out = _sc_gather(ids, word_embeddings)
    return out.reshape(input_ids.shape + (word_embeddings.shape[-1],))
